# baseline (speedup 1.0000x reference)
import functools
import jax
import jax.numpy as jnp
from jax import lax
from jax.experimental import pallas as pl
from jax.experimental.pallas import tpu as pltpu
from jax.experimental.pallas import tpu_sc as plsc

_B, _L, _E, _D = 4, 128, 256, 16
_NW = 32
_WPG = _NW // _B
_K = 8
_RPW = (_K * _B) // _NW
_TI = 8
_EPS = 1e-8


def _sc_body(ea_hbm, w_hbm, bias_hbm, paths_hbm, out_hbm,
             ea_v, w_v, bias_v, srep_v, buf_a, buf_b, out_v, sem_a, sem_b):
    wid = lax.axis_index("s") * 2 + lax.axis_index("c")
    b = wid // _WPG
    i0 = (wid % _WPG) * _RPW

    pltpu.sync_copy(ea_hbm.at[pl.ds(b * _E * _D, _E * _D)], ea_v)
    pltpu.sync_copy(w_hbm, w_v)
    pltpu.sync_copy(bias_hbm, bias_v)
    wv = w_v[...]
    bv = bias_v[...]

    def s_body(e, carry):
        r = ea_v[pl.ds(e * _D, _D)]
        srep_v[pl.ds(e * 16, 16)] = jnp.full((16,), jnp.sum(r * wv)) + bv
        return carry

    lax.fori_loop(0, _E, s_body, 0)

    iota = lax.iota(jnp.int32, 16)
    jrows = [iota + jb * 16 for jb in range(8)]
    bufs = [buf_a, buf_b]
    sems = [sem_a, sem_b]
    nbuf = 2

    def start(t):
        return pltpu.async_copy(
            paths_hbm.at[b, i0 + t], bufs[t % nbuf], sems[t % nbuf])

    cps = [start(t) for t in range(min(nbuf, _RPW))]
    for t in range(_RPW):
        cps[t % nbuf].wait()
        if t + nbuf < _RPW:
            cps[t % nbuf] = start(t + nbuf)
        buf = bufs[t % nbuf]
        zero = jnp.zeros((16,), jnp.float32)

        def e_body(e, carry):
            ns, ds = carry
            sb = srep_v[pl.ds(e * 16, 16)]
            ev = jnp.full((16,), e, jnp.int32)
            nn = []
            nd = []
            for jb in range(8):
                v = plsc.load_gather(buf, [jrows[jb], ev])
                nn.append(ns[jb] + sb * v)
                nd.append(ds[jb] + v)
            return tuple(nn), tuple(nd)

        ns, ds = lax.fori_loop(
            0, _E, e_body, (tuple([zero] * 8), tuple([zero] * 8)))
        for jb in range(8):
            out_v[pl.ds(t * _L + jb * 16, 16)] = ns[jb] / (ds[jb] + _EPS)

    pltpu.sync_copy(out_v, out_hbm.at[pl.ds((b * _K + i0) * _L, _RPW * _L)])


@jax.jit
def _sc_call(ea_flat, w_flat, bias_vec, edge_paths):
    mesh = plsc.VectorSubcoreMesh(core_axis_name="c", subcore_axis_name="s")
    f = pl.kernel(
        _sc_body,
        out_type=jax.ShapeDtypeStruct((_B * _K * _L,), jnp.float32),
        mesh=mesh,
        compiler_params=pltpu.CompilerParams(
            needs_layout_passes=False, has_side_effects=False),
        cost_estimate=pl.CostEstimate(
            flops=3 * _B * _K * _L * _E,
            transcendentals=0,
            bytes_accessed=4 * _B * _K * _L * _E),
        scratch_types=[
            pltpu.VMEM((_E * _D,), jnp.float32),
            pltpu.VMEM((_D,), jnp.float32),
            pltpu.VMEM((16,), jnp.float32),
            pltpu.VMEM((_E * 16,), jnp.float32),
            pltpu.VMEM((_L, _E), jnp.float32),
            pltpu.VMEM((_L, _E), jnp.float32),
            pltpu.VMEM((_RPW * _L,), jnp.float32),
            pltpu.SemaphoreType.DMA,
            pltpu.SemaphoreType.DMA,
        ],
    )
    return f(ea_flat, w_flat, bias_vec, edge_paths)


def _tc_body(ea_ref, w_ref, b_ref, ep_ref, out_ref):
    s = jnp.sum(ea_ref[0] * w_ref[...], axis=1) + b_ref[0, 0]
    p = ep_ref[0]
    num = jax.lax.dot_general(
        p, s, (((2,), (0,)), ((), ())), preferred_element_type=jnp.float32
    )
    den = jnp.sum(p, axis=2)
    out_ref[0] = num / (den + _EPS)


def _tc_call(edge_attr, edge_paths, W, b):
    ea = edge_attr.reshape(_B, _E, _D)
    wr = W.reshape(1, _D)
    br = b.reshape(1, 1)
    grid = (_B, (_L - _K) // _TI)
    koff = _K // _TI
    return pl.pallas_call(
        _tc_body,
        grid=grid,
        in_specs=[
            pl.BlockSpec((1, _E, _D), lambda bi, ic: (bi, 0, 0)),
            pl.BlockSpec((1, _D), lambda bi, ic: (0, 0)),
            pl.BlockSpec((1, 1), lambda bi, ic: (0, 0)),
            pl.BlockSpec((1, _TI, _L, _E), lambda bi, ic: (bi, ic + koff, 0, 0)),
        ],
        out_specs=pl.BlockSpec((1, _TI, _L), lambda bi, ic: (bi, ic, 0)),
        out_shape=jax.ShapeDtypeStruct((_B, _L - _K, _L), jnp.float32),
    )(ea, wr, br, edge_paths)


def kernel(edge_attr, edge_paths, ptr, W, b):
    nB, nL, _, nE = edge_paths.shape
    out_sc = _sc_call(
        edge_attr.reshape(-1), W.reshape(-1),
        jnp.full((16,), b[0], jnp.float32), edge_paths[:, :_K])
    out_tc = _tc_call(edge_attr, edge_paths, W, b)
    return jnp.concatenate([out_sc.reshape(nB, _K, nL), out_tc], axis=1)


# speedup vs baseline: 2.3822x; 2.3822x over previous
import functools
import jax
import jax.numpy as jnp
from jax.experimental import pallas as pl

_EPS = 1e-8


def _body(ea_ref, w_ref, bias_ref, ep_ref, out_ref):
    s = jnp.sum(ea_ref[0] * w_ref[...], axis=1) + bias_ref[0, 0]
    p = ep_ref[0]
    num = jax.lax.dot_general(
        p, s, (((2,), (0,)), ((), ())), preferred_element_type=jnp.float32
    )
    den = jnp.sum(p, axis=2)
    out_ref[0] = num / (den + _EPS)


def kernel(edge_attr, edge_paths, ptr, W, b):
    nB, nL, _, nE = edge_paths.shape
    nD = edge_attr.shape[1]
    TI = 32
    ea = edge_attr.reshape(nB, nE, nD)
    wr = W.reshape(1, nD)
    br = b.reshape(1, 1)
    grid = (nB, nL // TI)
    out = pl.pallas_call(
        _body,
        grid=grid,
        in_specs=[
            pl.BlockSpec((1, nE, nD), lambda bi, ic: (bi, 0, 0)),
            pl.BlockSpec((1, nD), lambda bi, ic: (0, 0)),
            pl.BlockSpec((1, 1), lambda bi, ic: (0, 0)),
            pl.BlockSpec((1, TI, nL, nE), lambda bi, ic: (bi, ic, 0, 0)),
        ],
        out_specs=pl.BlockSpec((1, TI, nL), lambda bi, ic: (bi, ic, 0)),
        out_shape=jax.ShapeDtypeStruct((nB, nL, nL), jnp.float32),
    )(ea, wr, br, edge_paths)
    return out


# speedup vs baseline: 2.7047x; 1.1354x over previous
import functools
import jax
import jax.numpy as jnp
from jax.experimental import pallas as pl

_EPS = 1e-8


def _body(ea_ref, w_ref, bias_ref, ep_ref, out_ref):
    s = jnp.sum(ea_ref[0] * w_ref[...], axis=1) + bias_ref[0, 0]
    p = ep_ref[0]
    num = jax.lax.dot_general(
        p, s, (((2,), (0,)), ((), ())), preferred_element_type=jnp.float32
    )
    den = jnp.sum(p, axis=2)
    out_ref[0] = num / (den + _EPS)


def kernel(edge_attr, edge_paths, ptr, W, b):
    nB, nL, _, nE = edge_paths.shape
    nD = edge_attr.shape[1]
    TI = 64
    ea = edge_attr.reshape(nB, nE, nD)
    wr = W.reshape(1, nD)
    br = b.reshape(1, 1)
    grid = (nB, nL // TI)
    out = pl.pallas_call(
        _body,
        grid=grid,
        in_specs=[
            pl.BlockSpec((1, nE, nD), lambda bi, ic: (bi, 0, 0)),
            pl.BlockSpec((1, nD), lambda bi, ic: (0, 0)),
            pl.BlockSpec((1, 1), lambda bi, ic: (0, 0)),
            pl.BlockSpec((1, TI, nL, nE), lambda bi, ic: (bi, ic, 0, 0)),
        ],
        out_specs=pl.BlockSpec((1, TI, nL), lambda bi, ic: (bi, ic, 0)),
        out_shape=jax.ShapeDtypeStruct((nB, nL, nL), jnp.float32),
    )(ea, wr, br, edge_paths)
    return out
